# Initial kernel scaffold; baseline (speedup 1.0000x reference)
import functools

import jax
import jax.numpy as jnp
from jax import lax
from jax.experimental import pallas as pl
from jax.experimental.pallas import tpu as pltpu
from jax.experimental.pallas import tpu_sc as plsc

N = 10000
D = 128
E = 320000

NC = 2
NS = 16
NW = NC * NS
NPAD = 10240
EPAD = 327680
EPW = EPAD // NW
CH = 128
NCHUNK = EPW // CH
ROWS_PER_TILE = NPAD // NS
FLUSH_STEPS = ROWS_PER_TILE // CH


def _stage1_body(x_ref, wg_ref, bg_ref, wy_ref, by_ref, g_ref, y_ref):
    xb = x_ref[...]
    g_ref[...] = jnp.dot(xb, wg_ref[...],
                         preferred_element_type=jnp.float32) + bg_ref[...]
    y_ref[...] = jnp.dot(xb, wy_ref[...],
                         preferred_element_type=jnp.float32) + by_ref[...] + xb


def _stage1(x, wg, bg, wy, by):
    br = 400
    grid = (N // br,)
    blk_x = pl.BlockSpec((br, D), lambda i: (i, 0))
    blk_w = pl.BlockSpec((D, D), lambda i: (0, 0))
    blk_b = pl.BlockSpec((1, D), lambda i: (0, 0))
    return pl.pallas_call(
        _stage1_body,
        grid=grid,
        in_specs=[blk_x, blk_w, blk_b, blk_w, blk_b],
        out_specs=[blk_x, blk_x],
        out_shape=[jax.ShapeDtypeStruct((N, D), jnp.float32),
                   jax.ShapeDtypeStruct((N, D), jnp.float32)],
    )(x, wg, bg, wy, by)


def _sc_body(x_hbm, s2_hbm, e2_hbm, a_hbm, b_hbm, z_hbm, out_hbm,
             a_v, b_v, sidx_v, eidx_v, rows_v, e_v, acc_sh, sem):
    cid = lax.axis_index("c")
    sid = lax.axis_index("s")
    wid = sid * NC + cid

    pltpu.sync_copy(a_hbm, a_v)
    pltpu.sync_copy(b_hbm, b_v)

    row0 = sid * ROWS_PER_TILE
    pltpu.sync_copy(z_hbm, rows_v)
    for t in range(FLUSH_STEPS):
        pltpu.sync_copy(rows_v, acc_sh.at[pl.ds(row0 + t * CH, CH)])
    plsc.subcore_barrier()

    crow0 = wid * NCHUNK

    def chunk_body(ci, carry):
        crow = crow0 + ci
        pltpu.sync_copy(s2_hbm.at[pl.ds(crow, 1)], sidx_v)
        pltpu.sync_copy(e2_hbm.at[pl.ds(crow, 1)], eidx_v)
        pltpu.async_copy(x_hbm.at[sidx_v.at[0]], rows_v, sem).wait()

        for g in range(8):
            sv = sidx_v[0, pl.ds(16 * g, 16)]
            ev = eidx_v[0, pl.ds(16 * g, 16)]
            av = plsc.load_gather(a_v, [sv])
            bv = plsc.load_gather(b_v, [ev])
            gate = 1.0 / (1.0 + jnp.exp(-(av + bv)))
            e_v[pl.ds(16 * g, 16)] = gate

        def srow(k, c):
            es = e_v[k]
            for j in range(8):
                sl = pl.ds(16 * j, 16)
                rows_v[k, sl] = rows_v[k, sl] * es
            return c

        lax.fori_loop(0, CH, srow, 0)

        pltpu.sync_copy(rows_v, acc_sh.at[eidx_v.at[0]], add=True)
        return carry

    lax.fori_loop(0, NCHUNK, chunk_body, 0)
    plsc.subcore_barrier()

    pltpu.sync_copy(acc_sh.at[pl.ds(row0, ROWS_PER_TILE)],
                    out_hbm.at[cid].at[pl.ds(row0, ROWS_PER_TILE)])


_sc_kernel = pl.kernel(
    _sc_body,
    out_type=jax.ShapeDtypeStruct((NC, NPAD, D), jnp.float32),
    mesh=plsc.VectorSubcoreMesh(core_axis_name="c", subcore_axis_name="s"),
    scratch_types=[
        pltpu.VMEM((NPAD,), jnp.float32),
        pltpu.VMEM((NPAD,), jnp.float32),
        pltpu.VMEM((1, CH), jnp.int32),
        pltpu.VMEM((1, CH), jnp.int32),
        pltpu.VMEM((CH, D), jnp.float32),
        pltpu.VMEM((CH,), jnp.float32),
        pltpu.VMEM_SHARED((NPAD, D), jnp.float32),
        pltpu.SemaphoreType.DMA,
    ],
)


def _stage3_body(m0_ref, m1_ref, wm_ref, y_ref, o_ref):
    mb = m0_ref[...] + m1_ref[...]
    o_ref[...] = jnp.dot(mb, wm_ref[...],
                         preferred_element_type=jnp.float32) + y_ref[...]


def _stage3(m0, m1, wm, y):
    br = 400
    grid = (N // br,)
    blk = pl.BlockSpec((br, D), lambda i: (i, 0))
    blk_w = pl.BlockSpec((D, D), lambda i: (0, 0))
    return pl.pallas_call(
        _stage3_body,
        grid=grid,
        in_specs=[blk, blk, blk_w, blk],
        out_specs=blk,
        out_shape=jax.ShapeDtypeStruct((N, D), jnp.float32),
    )(m0, m1, wm, y)


@jax.jit
def kernel(x, start, end, We, be, Wn, bn):
    x = x.astype(jnp.float32)
    s32 = start.astype(jnp.int32)
    e32 = end.astype(jnp.int32)

    wg = jnp.concatenate(
        [We[:D], We[D:], jnp.zeros((D, D - 2), jnp.float32)], axis=1)
    bg = jnp.concatenate([be, jnp.zeros((D - 1,), jnp.float32)])[None, :]
    wy = Wn[D:]
    by = bn[None, :]
    wm = Wn[:D]

    g_out, y = _stage1(x, wg, bg, wy, by)
    a = g_out[:, 0]
    b = g_out[:, 1]
    a_pad = jnp.concatenate([a, jnp.zeros((NPAD - N,), jnp.float32)])
    b_pad = jnp.concatenate([b, jnp.zeros((NPAD - N,), jnp.float32)])

    s_pad = jnp.concatenate(
        [s32, jnp.zeros((EPAD - E,), jnp.int32)]).reshape(EPAD // CH, CH)
    e_pad = jnp.concatenate(
        [e32, jnp.full((EPAD - E,), N, jnp.int32)]).reshape(EPAD // CH, CH)

    zeros_chunk = jnp.zeros((CH, D), jnp.float32)
    partials = _sc_kernel(x, s_pad, e_pad, a_pad, b_pad, zeros_chunk)

    return _stage3(partials[0, :N], partials[1, :N], wm, y)


# speedup vs baseline: 4.1777x; 4.1777x over previous
import functools

import jax
import jax.numpy as jnp
from jax import lax
from jax.experimental import pallas as pl
from jax.experimental.pallas import tpu as pltpu
from jax.experimental.pallas import tpu_sc as plsc

N = 10000
D = 128
E = 320000

NC = 2
NS = 16
NW = NC * NS
NPAD = 10240
EPAD = 327680
EPW = EPAD // NW
CH = 128
NCHUNK = EPW // CH
ROWS_PER_TILE = NPAD // NS
FLUSH_STEPS = ROWS_PER_TILE // CH


def _stage1_body(x_ref, wg_ref, bg_ref, wy_ref, by_ref, g_ref, y_ref):
    xb = x_ref[...]
    g_ref[...] = jnp.dot(xb, wg_ref[...],
                         preferred_element_type=jnp.float32) + bg_ref[...]
    y_ref[...] = jnp.dot(xb, wy_ref[...],
                         preferred_element_type=jnp.float32) + by_ref[...] + xb


def _stage1(x, wg, bg, wy, by):
    br = 400
    grid = (N // br,)
    blk_x = pl.BlockSpec((br, D), lambda i: (i, 0))
    blk_w = pl.BlockSpec((D, D), lambda i: (0, 0))
    blk_b = pl.BlockSpec((1, D), lambda i: (0, 0))
    return pl.pallas_call(
        _stage1_body,
        grid=grid,
        in_specs=[blk_x, blk_w, blk_b, blk_w, blk_b],
        out_specs=[blk_x, blk_x],
        out_shape=[jax.ShapeDtypeStruct((N, D), jnp.float32),
                   jax.ShapeDtypeStruct((N, D), jnp.float32)],
    )(x, wg, bg, wy, by)


def _sc_body(x_hbm, s2_hbm, e2_hbm, a_hbm, b_hbm, z_hbm, out_hbm,
             a_v, b_v, sidx_v, eidx_v, rows_v, e_v, acc_sh, sem):
    cid = lax.axis_index("c")
    sid = lax.axis_index("s")
    wid = sid * NC + cid

    pltpu.sync_copy(a_hbm, a_v)
    pltpu.sync_copy(b_hbm, b_v)

    row0 = sid * ROWS_PER_TILE
    pltpu.sync_copy(z_hbm, rows_v)
    for t in range(FLUSH_STEPS):
        pltpu.sync_copy(rows_v, acc_sh.at[pl.ds(row0 + t * CH, CH)])
    plsc.subcore_barrier()

    crow0 = wid * NCHUNK

    def chunk_body(ci, carry):
        crow = crow0 + ci
        pltpu.sync_copy(s2_hbm.at[pl.ds(crow, 1)], sidx_v)
        pltpu.sync_copy(e2_hbm.at[pl.ds(crow, 1)], eidx_v)
        gather = pltpu.async_copy(x_hbm.at[sidx_v.at[0]], rows_v, sem)

        for g in range(8):
            sv = sidx_v[0, pl.ds(16 * g, 16)]
            ev = eidx_v[0, pl.ds(16 * g, 16)]
            av = plsc.load_gather(a_v, [sv])
            bv = plsc.load_gather(b_v, [ev])
            gate = 1.0 / (1.0 + jnp.exp(-(av + bv)))
            e_v[pl.ds(16 * g, 16)] = gate

        gather.wait()

        def sgroup(g, c):
            e16 = e_v[pl.ds(16 * g, 16)]
            k0 = 16 * g
            for l in range(16):
                es = e16[l]
                for j in range(8):
                    sl = pl.ds(16 * j, 16)
                    rows_v[k0 + l, sl] = rows_v[k0 + l, sl] * es
            return c

        lax.fori_loop(0, 8, sgroup, 0)

        pltpu.sync_copy(rows_v, acc_sh.at[eidx_v.at[0]], add=True)
        return carry

    lax.fori_loop(0, NCHUNK, chunk_body, 0)
    plsc.subcore_barrier()

    pltpu.sync_copy(acc_sh.at[pl.ds(row0, ROWS_PER_TILE)],
                    out_hbm.at[cid].at[pl.ds(row0, ROWS_PER_TILE)])


_sc_kernel = pl.kernel(
    _sc_body,
    out_type=jax.ShapeDtypeStruct((NC, NPAD, D), jnp.float32),
    mesh=plsc.VectorSubcoreMesh(core_axis_name="c", subcore_axis_name="s"),
    compiler_params=pltpu.CompilerParams(needs_layout_passes=False),
    scratch_types=[
        pltpu.VMEM((NPAD,), jnp.float32),
        pltpu.VMEM((NPAD,), jnp.float32),
        pltpu.VMEM((1, CH), jnp.int32),
        pltpu.VMEM((1, CH), jnp.int32),
        pltpu.VMEM((CH, D), jnp.float32),
        pltpu.VMEM((CH,), jnp.float32),
        pltpu.VMEM_SHARED((NPAD, D), jnp.float32),
        pltpu.SemaphoreType.DMA,
    ],
)


def _stage3_body(m0_ref, m1_ref, wm_ref, y_ref, o_ref):
    mb = m0_ref[...] + m1_ref[...]
    o_ref[...] = jnp.dot(mb, wm_ref[...],
                         preferred_element_type=jnp.float32) + y_ref[...]


def _stage3(m0, m1, wm, y):
    br = 400
    grid = (N // br,)
    blk = pl.BlockSpec((br, D), lambda i: (i, 0))
    blk_w = pl.BlockSpec((D, D), lambda i: (0, 0))
    return pl.pallas_call(
        _stage3_body,
        grid=grid,
        in_specs=[blk, blk, blk_w, blk],
        out_specs=blk,
        out_shape=jax.ShapeDtypeStruct((N, D), jnp.float32),
    )(m0, m1, wm, y)


@jax.jit
def kernel(x, start, end, We, be, Wn, bn):
    x = x.astype(jnp.float32)
    s32 = start.astype(jnp.int32)
    e32 = end.astype(jnp.int32)

    wg = jnp.concatenate(
        [We[:D], We[D:], jnp.zeros((D, D - 2), jnp.float32)], axis=1)
    bg = jnp.concatenate([be, jnp.zeros((D - 1,), jnp.float32)])[None, :]
    wy = Wn[D:]
    by = bn[None, :]
    wm = Wn[:D]

    g_out, y = _stage1(x, wg, bg, wy, by)
    a = g_out[:, 0]
    b = g_out[:, 1]
    a_pad = jnp.concatenate([a, jnp.zeros((NPAD - N,), jnp.float32)])
    b_pad = jnp.concatenate([b, jnp.zeros((NPAD - N,), jnp.float32)])

    s_pad = jnp.concatenate(
        [s32, jnp.zeros((EPAD - E,), jnp.int32)]).reshape(EPAD // CH, CH)
    e_pad = jnp.concatenate(
        [e32, jnp.full((EPAD - E,), N, jnp.int32)]).reshape(EPAD // CH, CH)

    zeros_chunk = jnp.zeros((CH, D), jnp.float32)
    partials = _sc_kernel(x, s_pad, e_pad, a_pad, b_pad, zeros_chunk)

    return _stage3(partials[0, :N], partials[1, :N], wm, y)


# speedup vs baseline: 4.3397x; 1.0388x over previous
import functools

import jax
import jax.numpy as jnp
from jax import lax
from jax.experimental import pallas as pl
from jax.experimental.pallas import tpu as pltpu
from jax.experimental.pallas import tpu_sc as plsc

N = 10000
D = 128
E = 320000

NC = 2
NS = 16
NW = NC * NS
NPAD = 10240
CH = 64
NCHUNK = 162
EPW = CH * NCHUNK
EPAD = EPW * NW
ROWS_PER_TILE = NPAD // NS
NB = 3


def _stage1_body(x_ref, wg_ref, bg_ref, wy_ref, by_ref, g_ref, y_ref):
    xb = x_ref[...]
    g_ref[...] = jnp.dot(xb, wg_ref[...],
                         preferred_element_type=jnp.float32) + bg_ref[...]
    y_ref[...] = jnp.dot(xb, wy_ref[...],
                         preferred_element_type=jnp.float32) + by_ref[...] + xb


def _stage1(x, wg, bg, wy, by):
    br = 400
    grid = (N // br,)
    blk_x = pl.BlockSpec((br, D), lambda i: (i, 0))
    blk_w = pl.BlockSpec((D, D), lambda i: (0, 0))
    blk_b = pl.BlockSpec((1, D), lambda i: (0, 0))
    return pl.pallas_call(
        _stage1_body,
        grid=grid,
        in_specs=[blk_x, blk_w, blk_b, blk_w, blk_b],
        out_specs=[blk_x, blk_x],
        out_shape=[jax.ShapeDtypeStruct((N, D), jnp.float32),
                   jax.ShapeDtypeStruct((N, D), jnp.float32)],
    )(x, wg, bg, wy, by)


def _sc_body(x_hbm, s2_hbm, e2_hbm, a_hbm, b_hbm, z_hbm, out_hbm,
             a_v, b_v, sidx_v, eidx_v, rows_v, e_v, acc_sh, sem_g, sem_s):
    cid = lax.axis_index("c")
    sid = lax.axis_index("s")
    wid = sid * NC + cid

    pltpu.sync_copy(a_hbm, a_v)
    pltpu.sync_copy(b_hbm, b_v)

    row0 = sid * ROWS_PER_TILE
    pltpu.sync_copy(z_hbm.at[pl.ds(row0, ROWS_PER_TILE)],
                    acc_sh.at[pl.ds(row0, ROWS_PER_TILE)])
    plsc.subcore_barrier()

    crow0 = wid * NCHUNK

    def issue_gather(b, crow):
        pltpu.sync_copy(s2_hbm.at[pl.ds(crow, 1)], sidx_v.at[b])
        pltpu.sync_copy(e2_hbm.at[pl.ds(crow, 1)], eidx_v.at[b])
        pltpu.async_copy(x_hbm.at[sidx_v.at[b].at[0]], rows_v.at[b],
                         sem_g.at[b])

    issue_gather(0, crow0)
    issue_gather(1, crow0 + 1)

    def visit(b, ci):
        br = (b + 2) % NB

        pltpu.make_async_copy(x_hbm.at[sidx_v.at[b].at[0]], rows_v.at[b],
                              sem_g.at[b]).wait()

        for g in range(CH // 16):
            sv = sidx_v[b, 0, pl.ds(16 * g, 16)]
            ev = eidx_v[b, 0, pl.ds(16 * g, 16)]
            av = plsc.load_gather(a_v, [sv])
            bv = plsc.load_gather(b_v, [ev])
            gate = 1.0 / (1.0 + jnp.exp(-(av + bv)))
            e_v[pl.ds(16 * g, 16)] = gate

        def sgroup(g, c):
            e16 = e_v[pl.ds(16 * g, 16)]
            k0 = 16 * g
            for l in range(16):
                es = e16[l]
                for j in range(8):
                    sl = pl.ds(16 * j, 16)
                    rows_v[b, k0 + l, sl] = rows_v[b, k0 + l, sl] * es
            return c

        lax.fori_loop(0, CH // 16, sgroup, 0)

        pltpu.async_copy(rows_v.at[b], acc_sh.at[eidx_v.at[b].at[0]],
                         sem_s.at[b], add=True)

        @pl.when(ci >= 1)
        def _():
            pltpu.make_async_copy(rows_v.at[br],
                                  acc_sh.at[eidx_v.at[br].at[0]],
                                  sem_s.at[br]).wait()

        @pl.when(ci + 2 < NCHUNK)
        def _():
            issue_gather(br, crow0 + ci + 2)

    def triple_body(i3, carry):
        c0 = 3 * i3
        visit(0, c0)
        visit(1, c0 + 1)
        visit(2, c0 + 2)
        return carry

    lax.fori_loop(0, NCHUNK // NB, triple_body, 0)

    bl = (NCHUNK - 1) % NB
    pltpu.make_async_copy(rows_v.at[bl], acc_sh.at[eidx_v.at[bl].at[0]],
                          sem_s.at[bl]).wait()
    plsc.subcore_barrier()

    pltpu.sync_copy(acc_sh.at[pl.ds(row0, ROWS_PER_TILE)],
                    out_hbm.at[cid].at[pl.ds(row0, ROWS_PER_TILE)])


_sc_kernel = pl.kernel(
    _sc_body,
    out_type=jax.ShapeDtypeStruct((NC, NPAD, D), jnp.float32),
    mesh=plsc.VectorSubcoreMesh(core_axis_name="c", subcore_axis_name="s"),
    compiler_params=pltpu.CompilerParams(needs_layout_passes=False),
    scratch_types=[
        pltpu.VMEM((NPAD,), jnp.float32),
        pltpu.VMEM((NPAD,), jnp.float32),
        pltpu.VMEM((NB, 1, CH), jnp.int32),
        pltpu.VMEM((NB, 1, CH), jnp.int32),
        pltpu.VMEM((NB, CH, D), jnp.float32),
        pltpu.VMEM((CH,), jnp.float32),
        pltpu.VMEM_SHARED((NPAD, D), jnp.float32),
        pltpu.SemaphoreType.DMA((NB,)),
        pltpu.SemaphoreType.DMA((NB,)),
    ],
)


def _stage3_body(m0_ref, m1_ref, wm_ref, y_ref, o_ref):
    mb = m0_ref[...] + m1_ref[...]
    o_ref[...] = jnp.dot(mb, wm_ref[...],
                         preferred_element_type=jnp.float32) + y_ref[...]


def _stage3(m0, m1, wm, y):
    br = 400
    grid = (N // br,)
    blk = pl.BlockSpec((br, D), lambda i: (i, 0))
    blk_w = pl.BlockSpec((D, D), lambda i: (0, 0))
    return pl.pallas_call(
        _stage3_body,
        grid=grid,
        in_specs=[blk, blk, blk_w, blk],
        out_specs=blk,
        out_shape=jax.ShapeDtypeStruct((N, D), jnp.float32),
    )(m0, m1, wm, y)


@jax.jit
def kernel(x, start, end, We, be, Wn, bn):
    x = x.astype(jnp.float32)
    s32 = start.astype(jnp.int32)
    e32 = end.astype(jnp.int32)

    wg = jnp.concatenate(
        [We[:D], We[D:], jnp.zeros((D, D - 2), jnp.float32)], axis=1)
    bg = jnp.concatenate([be, jnp.zeros((D - 1,), jnp.float32)])[None, :]
    wy = Wn[D:]
    by = bn[None, :]
    wm = Wn[:D]

    g_out, y = _stage1(x, wg, bg, wy, by)
    a = g_out[:, 0]
    b = g_out[:, 1]
    a_pad = jnp.concatenate([a, jnp.zeros((NPAD - N,), jnp.float32)])
    b_pad = jnp.concatenate([b, jnp.zeros((NPAD - N,), jnp.float32)])

    s_pad = jnp.concatenate(
        [s32, jnp.zeros((EPAD - E,), jnp.int32)]).reshape(EPAD // CH, CH)
    e_pad = jnp.concatenate(
        [e32, jnp.full((EPAD - E,), N, jnp.int32)]).reshape(EPAD // CH, CH)

    zeros_nodes = jnp.zeros((NPAD, D), jnp.float32)
    partials = _sc_kernel(x, s_pad, e_pad, a_pad, b_pad, zeros_nodes)

    return _stage3(partials[0, :N], partials[1, :N], wm, y)
